# baseline (speedup 1.0000x reference)
import functools

import jax
import jax.numpy as jnp
from jax import lax
from jax.experimental import pallas as pl
from jax.experimental.pallas import tpu as pltpu
from jax.experimental.pallas import tpu_sc as plsc

_N_EDGES = 320000
_D_EDGE = 16
_E_WORDS = _N_EDGES * _D_EDGE
_NC = 2
_NS = 16
_NW = _NC * _NS
_WORDS_PER_W = _E_WORDS // _NW
_CHUNK = 16000
_NCHUNK = _WORDS_PER_W // _CHUNK


@functools.partial(
    pl.kernel,
    mesh=plsc.VectorSubcoreMesh(core_axis_name="c", subcore_axis_name="s"),
    out_type=jax.ShapeDtypeStruct((_N_EDGES, _D_EDGE), jnp.float32),
    scratch_types=[
        pltpu.VMEM((_CHUNK // _D_EDGE, _D_EDGE), jnp.float32),
    ],
)
def _edge_copy_sc(edge_hbm, out_hbm, buf):
    wid = lax.axis_index("s") * _NC + lax.axis_index("c")
    base = wid * _WORDS_PER_W
    rows_per_w = _N_EDGES // _NW
    chunk_rows = _CHUNK // _D_EDGE
    base_row = wid * rows_per_w
    for k in range(rows_per_w // chunk_rows):
        r0 = base_row + k * chunk_rows
        pltpu.sync_copy(edge_hbm.at[pl.ds(r0, chunk_rows), :], buf)
        pltpu.sync_copy(buf, out_hbm.at[pl.ds(r0, chunk_rows), :])


def _node_copy_body(node_ref, node_out_ref):
    node_out_ref[...] = node_ref[...]


def kernel(node_feats, edge_index, edge_attr):
    n_nodes, d_feat = node_feats.shape
    grid = 10
    nb = n_nodes // grid
    node_out = pl.pallas_call(
        _node_copy_body,
        grid=(grid,),
        in_specs=[pl.BlockSpec((nb, d_feat), lambda i: (i, 0))],
        out_specs=pl.BlockSpec((nb, d_feat), lambda i: (i, 0)),
        out_shape=jax.ShapeDtypeStruct((n_nodes, d_feat), node_feats.dtype),
    )(node_feats)
    edge_out = _edge_copy_sc(edge_attr)
    return (node_out, edge_out)


# speedup vs baseline: 1.0017x; 1.0017x over previous
import functools

import jax
import jax.numpy as jnp
from jax import lax
from jax.experimental import pallas as pl
from jax.experimental.pallas import tpu as pltpu
from jax.experimental.pallas import tpu_sc as plsc

_N_EDGES = 320000
_D_EDGE = 16
_NC = 2
_NS = 16
_NW = _NC * _NS
_ROWS_PER_W = _N_EDGES // _NW
_CHUNK = 400
_NCHUNK = _ROWS_PER_W // _CHUNK


@functools.partial(
    pl.kernel,
    mesh=plsc.VectorSubcoreMesh(core_axis_name="c", subcore_axis_name="s"),
    out_type=jax.ShapeDtypeStruct((_N_EDGES, _D_EDGE), jnp.float32),
    scratch_types=[
        pltpu.VMEM((_CHUNK, _D_EDGE), jnp.float32),
        pltpu.VMEM((_CHUNK, _D_EDGE), jnp.float32),
        pltpu.SemaphoreType.DMA,
        pltpu.SemaphoreType.DMA,
        pltpu.SemaphoreType.DMA,
        pltpu.SemaphoreType.DMA,
    ],
)
def _edge_copy_sc(edge_hbm, out_hbm, buf0, buf1, gs0, gs1, ss0, ss1):
    wid = lax.axis_index("s") * _NC + lax.axis_index("c")
    base = wid * _ROWS_PER_W
    bufs = (buf0, buf1)
    gsems = (gs0, gs1)
    ssems = (ss0, ss1)

    def src(k):
        return edge_hbm.at[pl.ds(base + k * _CHUNK, _CHUNK), :]

    def dst(k):
        return out_hbm.at[pl.ds(base + k * _CHUNK, _CHUNK), :]

    gathers = [None] * _NCHUNK
    scatters = [None] * _NCHUNK
    gathers[0] = pltpu.async_copy(src(0), bufs[0], gsems[0])
    for k in range(_NCHUNK):
        b = k % 2
        gathers[k].wait()
        scatters[k] = pltpu.async_copy(bufs[b], dst(k), ssems[b])
        if k + 1 < _NCHUNK:
            if k >= 1:
                scatters[k - 1].wait()
            nb = (k + 1) % 2
            gathers[k + 1] = pltpu.async_copy(src(k + 1), bufs[nb], gsems[nb])
    scatters[_NCHUNK - 2].wait()
    scatters[_NCHUNK - 1].wait()


def _node_copy_body(node_ref, node_out_ref):
    node_out_ref[...] = node_ref[...]


def kernel(node_feats, edge_index, edge_attr):
    n_nodes, d_feat = node_feats.shape
    grid = 10
    nb = n_nodes // grid
    node_out = pl.pallas_call(
        _node_copy_body,
        grid=(grid,),
        in_specs=[pl.BlockSpec((nb, d_feat), lambda i: (i, 0))],
        out_specs=pl.BlockSpec((nb, d_feat), lambda i: (i, 0)),
        out_shape=jax.ShapeDtypeStruct((n_nodes, d_feat), node_feats.dtype),
    )(node_feats)
    edge_out = _edge_copy_sc(edge_attr)
    return (node_out, edge_out)
